# Initial kernel scaffold; baseline (speedup 1.0000x reference)
import jax
import jax.numpy as jnp
from jax.experimental import pallas as pl


def kernel(x, h_node, batch, size, W_rat, b_rat, W_gate, b_gate):
    raise NotImplementedError("write your pallas kernel here")


# speedup vs baseline: 7.8041x; 7.8041x over previous
import functools

import jax
import jax.numpy as jnp
from jax.experimental import pallas as pl
from jax.experimental.pallas import tpu as pltpu

_S = 512
_D = 128
_B = 2048


def _tc_body(x_ref, h_ref, b_ref, W_rat_ref, b_rat_ref, W_gate_ref,
             b_gate_ref, h_out_ref, c_out_ref, r_ref, env_ref, acc_ref,
             *, nb, n):
    pid = pl.program_id(0)

    xb = x_ref[...]
    hb = h_ref[...]
    seg = b_ref[0]

    feat = jnp.maximum(
        jnp.dot(xb, W_rat_ref[...], preferred_element_type=jnp.float32)
        + b_rat_ref[...], 0.0)
    gpre = jnp.dot(feat, W_gate_ref[...],
                   preferred_element_type=jnp.float32) + b_gate_ref[0, 0]
    g = jax.nn.sigmoid(gpre)

    row = pid * _B + jax.lax.broadcasted_iota(jnp.int32, (_B, 1), 0)
    valid = row < n
    g = jnp.where(valid, g, 0.0)
    hm = jnp.where(valid, hb, 0.0)

    lane = jax.lax.broadcasted_iota(jnp.int32, (_B, _D), 1)
    extra = jnp.where(lane == 0, g, jnp.where(
        lane == 1, jnp.where(valid, 1.0, 0.0), 0.0))
    V = jnp.concatenate([g * hm, hm, extra], axis=1)

    ohT = (jax.lax.broadcasted_iota(jnp.int32, (_S, _B), 0)
           == seg).astype(jnp.float32)

    @pl.when(pid == 0)
    def _():
        acc_ref[...] = jnp.zeros_like(acc_ref)

    acc_ref[...] += jnp.dot(ohT, V, preferred_element_type=jnp.float32)

    @pl.when(pid == nb - 1)
    def _():
        acc = acc_ref[...]
        h_out_ref[...] = acc[:, :_D]
        c_out_ref[...] = acc[:, _D:2 * _D] - acc[:, :_D]
        r_ref[...] = acc[:, 2 * _D:2 * _D + 1] + 1e-8
        env_ref[...] = (acc[:, 2 * _D + 1:2 * _D + 2]
                        - acc[:, 2 * _D:2 * _D + 1]) + 1e-8


def kernel(x, h_node, batch, size, W_rat, b_rat, W_gate, b_gate):
    n, d = x.shape
    nb = (n + _B - 1) // _B
    npad = nb * _B

    seg = batch.astype(jnp.int32)
    seg = jnp.pad(seg, (0, npad - n), constant_values=_S).reshape(nb, 1, _B)

    grid = (nb,)
    out = pl.pallas_call(
        functools.partial(_tc_body, nb=nb, n=n),
        grid=grid,
        in_specs=[
            pl.BlockSpec((_B, d), lambda i: (i, 0)),
            pl.BlockSpec((_B, d), lambda i: (i, 0)),
            pl.BlockSpec((1, 1, _B), lambda i: (i, 0, 0)),
            pl.BlockSpec((d, d), lambda i: (0, 0)),
            pl.BlockSpec((1, d), lambda i: (0, 0)),
            pl.BlockSpec((d, 1), lambda i: (0, 0)),
            pl.BlockSpec((1, 1), lambda i: (0, 0)),
        ],
        out_specs=[
            pl.BlockSpec((_S, _D), lambda i: (0, 0)),
            pl.BlockSpec((_S, _D), lambda i: (0, 0)),
            pl.BlockSpec((_S, 1), lambda i: (0, 0)),
            pl.BlockSpec((_S, 1), lambda i: (0, 0)),
        ],
        out_shape=[
            jax.ShapeDtypeStruct((_S, _D), jnp.float32),
            jax.ShapeDtypeStruct((_S, _D), jnp.float32),
            jax.ShapeDtypeStruct((_S, 1), jnp.float32),
            jax.ShapeDtypeStruct((_S, 1), jnp.float32),
        ],
        scratch_shapes=[pltpu.VMEM((_S, 3 * _D), jnp.float32)],
    )(x, h_node, seg, W_rat, b_rat.reshape(1, d), W_gate,
      b_gate.reshape(1, 1))
    return tuple(out)
